# baseline (speedup 1.0000x reference)
import functools

import jax
import jax.numpy as jnp
from jax import lax
from jax.experimental import pallas as pl
from jax.experimental.pallas import tpu as pltpu
from jax.experimental.pallas import tpu_sc as plsc

C, T, H, W = 3, 64, 384, 384
TS = T // 4
FRAME = H * W
QUARTER = FRAME // 4
NW = 32
FAST_PIECES = C * T * 4
SLOW_PIECES = C * TS * 4
FAST_PER_W = FAST_PIECES // NW
SLOW_PER_W = SLOW_PIECES // NW
PER_W = FAST_PER_W + SLOW_PER_W


def _sc_pack(frames_flat):
    mesh = plsc.VectorSubcoreMesh(core_axis_name="c", subcore_axis_name="s")

    @functools.partial(
        pl.kernel,
        mesh=mesh,
        out_type=(
            jax.ShapeDtypeStruct((C * TS * FRAME,), jnp.float32),
            jax.ShapeDtypeStruct((C * T * FRAME,), jnp.float32),
        ),
        scratch_types=[
            pltpu.VMEM((2, QUARTER), jnp.float32),
            pltpu.SemaphoreType.DMA,
            pltpu.SemaphoreType.DMA,
        ],
    )
    def k(src, slow_out, fast_out, buf, sem_r, sem_w):
        wid = lax.axis_index("s") * 2 + lax.axis_index("c")

        def piece(j):
            if j < FAST_PER_W:
                p = wid * FAST_PER_W + j
                return p * QUARTER, fast_out, p * QUARTER
            p = wid * SLOW_PER_W + (j - FAST_PER_W)
            slab = p // 4
            q = p % 4
            c = slab // TS
            i = slab % TS
            src_off = (c * T + (i * 21) // 5) * FRAME + q * QUARTER
            return src_off, slow_out, p * QUARTER

        def rd(j):
            src_off, _, _ = piece(j)
            return pltpu.make_async_copy(
                src.at[pl.ds(src_off, QUARTER)], buf.at[j % 2], sem_r
            )

        def wr(j):
            _, dst, dst_off = piece(j)
            return pltpu.make_async_copy(
                buf.at[j % 2], dst.at[pl.ds(dst_off, QUARTER)], sem_w
            )

        rd(0).start()
        for j in range(PER_W):
            rd(j).wait()
            if j >= 1:
                wr(j - 1).wait()
            wr(j).start()
            if j + 1 < PER_W:
                rd(j + 1).start()
        wr(PER_W - 1).wait()

    return k(frames_flat)


def kernel(frames):
    slow, fast = _sc_pack(frames.reshape(-1))
    return (slow.reshape(C, TS, H, W), fast.reshape(C, T, H, W))


# speedup vs baseline: 3.3050x; 3.3050x over previous
import functools

import jax
import jax.numpy as jnp
from jax import lax
from jax.experimental import pallas as pl
from jax.experimental.pallas import tpu as pltpu
from jax.experimental.pallas import tpu_sc as plsc

C, T, H, W = 3, 64, 384, 384
TS = T // 4
QROWS = H // 4
NW = 32
FAST_PER_W = C * T * 4 // NW
SLOW_PER_W = C * TS * 4 // NW
PER_W = FAST_PER_W + SLOW_PER_W


def _sc_pack(frames):
    mesh = plsc.VectorSubcoreMesh(core_axis_name="c", subcore_axis_name="s")

    @functools.partial(
        pl.kernel,
        mesh=mesh,
        out_type=(
            jax.ShapeDtypeStruct((C, TS, H, W), jnp.float32),
            jax.ShapeDtypeStruct((C, T, H, W), jnp.float32),
        ),
        scratch_types=[
            pltpu.VMEM((2, QROWS, W), jnp.float32),
            pltpu.SemaphoreType.DMA,
            pltpu.SemaphoreType.DMA,
        ],
        compiler_params=pltpu.CompilerParams(use_tc_tiling_on_sc=True),
    )
    def k(src, slow_out, fast_out, buf, sem_r, sem_w):
        wid = lax.axis_index("s") * 2 + lax.axis_index("c")

        def piece(j):
            if j < FAST_PER_W:
                p = wid * FAST_PER_W + j
                c, t, q = p // (T * 4), (p // 4) % T, p % 4
                rows = pl.ds(q * QROWS, QROWS)
                return src.at[c, t, rows], fast_out.at[c, t, rows]
            p = wid * SLOW_PER_W + (j - FAST_PER_W)
            slab = p // 4
            q = p % 4
            c = slab // TS
            i = slab % TS
            t = (i * 21) // 5
            rows = pl.ds(q * QROWS, QROWS)
            return src.at[c, t, rows], slow_out.at[c, i, rows]

        def rd(j):
            s, _ = piece(j)
            return pltpu.make_async_copy(s, buf.at[j % 2], sem_r)

        def wr(j):
            _, d = piece(j)
            return pltpu.make_async_copy(buf.at[j % 2], d, sem_w)

        rd(0).start()
        for j in range(PER_W):
            rd(j).wait()
            if j >= 1:
                wr(j - 1).wait()
            wr(j).start()
            if j + 1 < PER_W:
                rd(j + 1).start()
        wr(PER_W - 1).wait()

    return k(frames)


def kernel(frames):
    return _sc_pack(frames)


# speedup vs baseline: 3.3472x; 1.0128x over previous
import functools

import jax
import jax.numpy as jnp
from jax import lax
from jax.experimental import pallas as pl
from jax.experimental.pallas import tpu as pltpu
from jax.experimental.pallas import tpu_sc as plsc

C, T, H, W = 3, 64, 384, 384
TS = T // 4
QROWS = H // 4
NW = 32
PER_W = C * T * 4 // NW


def _sc_pack(frames):
    mesh = plsc.VectorSubcoreMesh(core_axis_name="c", subcore_axis_name="s")

    @functools.partial(
        pl.kernel,
        mesh=mesh,
        out_type=(
            jax.ShapeDtypeStruct((C, TS, H, W), jnp.float32),
            jax.ShapeDtypeStruct((C, T, H, W), jnp.float32),
        ),
        scratch_types=[
            pltpu.VMEM((2, QROWS, W), jnp.float32),
            pltpu.SemaphoreType.DMA,
            pltpu.SemaphoreType.DMA,
            pltpu.SemaphoreType.DMA,
        ],
        compiler_params=pltpu.CompilerParams(use_tc_tiling_on_sc=True),
    )
    def k(src, slow_out, fast_out, buf, sem_r, sem_w, sem_w2):
        wid = lax.axis_index("s") * 2 + lax.axis_index("c")

        def coords(j):
            p = wid * PER_W + j
            return p // (T * 4), (p // 4) % T, p % 4

        def rd(j):
            c, t, q = coords(j)
            rows = pl.ds(q * QROWS, QROWS)
            return pltpu.make_async_copy(
                src.at[c, t, rows], buf.at[j % 2], sem_r
            )

        def wr(j):
            c, t, q = coords(j)
            rows = pl.ds(q * QROWS, QROWS)
            return pltpu.make_async_copy(
                buf.at[j % 2], fast_out.at[c, t, rows], sem_w
            )

        def slow_cond(j):
            _, t, _ = coords(j)
            i = (5 * t + 20) // 21
            return ((i * 21) // 5) == t

        def wr2(j):
            c, t, q = coords(j)
            i = (5 * t + 20) // 21
            rows = pl.ds(q * QROWS, QROWS)
            return pltpu.make_async_copy(
                buf.at[j % 2], slow_out.at[c, i, rows], sem_w2
            )

        rd(0).start()
        for j in range(PER_W):
            rd(j).wait()
            if j >= 1:
                wr(j - 1).wait()

                @pl.when(slow_cond(j - 1))
                def _():
                    wr2(j - 1).wait()

            wr(j).start()

            @pl.when(slow_cond(j))
            def _():
                wr2(j).start()

            if j + 1 < PER_W:
                rd(j + 1).start()
        wr(PER_W - 1).wait()

        @pl.when(slow_cond(PER_W - 1))
        def _():
            wr2(PER_W - 1).wait()

    return k(frames)


def kernel(frames):
    return _sc_pack(frames)


# speedup vs baseline: 3.4639x; 1.0349x over previous
import functools

import jax
import jax.numpy as jnp
from jax import lax
from jax.experimental import pallas as pl
from jax.experimental.pallas import tpu as pltpu
from jax.experimental.pallas import tpu_sc as plsc

C, T, H, W = 3, 64, 384, 384
TS = T // 4
PPF = 6
QROWS = H // PPF
NW = 32
PER_W = C * T * PPF // NW
NBUF = 4


def _sc_pack(frames):
    mesh = plsc.VectorSubcoreMesh(core_axis_name="c", subcore_axis_name="s")

    @functools.partial(
        pl.kernel,
        mesh=mesh,
        out_type=(
            jax.ShapeDtypeStruct((C, TS, H, W), jnp.float32),
            jax.ShapeDtypeStruct((C, T, H, W), jnp.float32),
        ),
        scratch_types=[
            pltpu.VMEM((NBUF, QROWS, W), jnp.float32),
            pltpu.SemaphoreType.DMA,
            pltpu.SemaphoreType.DMA,
            pltpu.SemaphoreType.DMA,
        ],
        compiler_params=pltpu.CompilerParams(use_tc_tiling_on_sc=True),
    )
    def k(src, slow_out, fast_out, buf, sem_r, sem_w, sem_w2):
        wid = lax.axis_index("s") * 2 + lax.axis_index("c")

        def coords(j):
            p = wid * PER_W + j
            return p // (T * PPF), (p // PPF) % T, p % PPF

        def rd(j):
            c, t, q = coords(j)
            rows = pl.ds(q * QROWS, QROWS)
            return pltpu.make_async_copy(
                src.at[c, t, rows], buf.at[j % NBUF], sem_r
            )

        def wr(j):
            c, t, q = coords(j)
            rows = pl.ds(q * QROWS, QROWS)
            return pltpu.make_async_copy(
                buf.at[j % NBUF], fast_out.at[c, t, rows], sem_w
            )

        def slow_cond(j):
            _, t, _ = coords(j)
            i = (5 * t + 20) // 21
            return ((i * 21) // 5) == t

        def wr2(j):
            c, t, q = coords(j)
            i = (5 * t + 20) // 21
            rows = pl.ds(q * QROWS, QROWS)
            return pltpu.make_async_copy(
                buf.at[j % NBUF], slow_out.at[c, i, rows], sem_w2
            )

        def wait_writes(j):
            wr(j).wait()

            @pl.when(slow_cond(j))
            def _():
                wr2(j).wait()

        rd(0).start()
        rd(1).start()
        for j in range(PER_W):
            rd(j).wait()
            if j >= 2:
                wait_writes(j - 2)
            wr(j).start()

            @pl.when(slow_cond(j))
            def _():
                wr2(j).start()

            if j + 2 < PER_W:
                rd(j + 2).start()
        wait_writes(PER_W - 2)
        wait_writes(PER_W - 1)

    return k(frames)


def kernel(frames):
    return _sc_pack(frames)
